# Initial kernel scaffold; baseline (speedup 1.0000x reference)
import jax
import jax.numpy as jnp
from jax.experimental import pallas as pl


def kernel(x, edge_index, W1, b1):
    raise NotImplementedError("write your pallas kernel here")


# speedup vs baseline: 21.9062x; 21.9062x over previous
import functools

import jax
import jax.numpy as jnp
from jax import lax
from jax.experimental import pallas as pl
from jax.experimental.pallas import tpu as pltpu
from jax.experimental.pallas import tpu_sc as plsc

N = 10000
E = 320000
F = 128
NC = 2
NS = 16
NW = NC * NS
PAD_N = 10240
STRIPE = PAD_N // NS
E_PER_W = E // NW
CHUNK = 128
N_FULL = E_PER_W // CHUNK
TAIL = E_PER_W - N_FULL * CHUNK

_MESH = plsc.VectorSubcoreMesh(
    core_axis_name="c", subcore_axis_name="s", num_cores=NC, num_subcores=NS
)


def _fill1d(ref, n, val):
    v = jnp.full((16,), val, dtype=ref.dtype)

    def body(i, _):
        ref[pl.ds(i * 16, 16)] = v
        return 0

    lax.fori_loop(0, n // 16, body, 0)


def _fill2d(ref, rows, val):
    v = jnp.full((16,), val, dtype=ref.dtype)

    def body(i, _):
        ref[i // 8, pl.ds((i % 8) * 16, 16)] = v
        return 0

    lax.fori_loop(0, rows * 8, body, 0)


def _hist_body(dst_hbm, out_hbm, idxb, onesb, i16, ones16, zb, hist):
    c = lax.axis_index("c")
    s = lax.axis_index("s")
    wid = s * NC + c
    _fill1d(onesb, CHUNK, 1.0)
    _fill1d(ones16, TAIL, 1.0)
    _fill1d(zb, STRIPE, 0.0)
    pltpu.sync_copy(zb, hist.at[pl.ds(s * STRIPE, STRIPE)])
    plsc.subcore_barrier()
    base = wid * E_PER_W

    def chunk(ci, _):
        off = base + ci * CHUNK
        pltpu.sync_copy(dst_hbm.at[pl.ds(off, CHUNK)], idxb)
        pltpu.sync_copy(onesb, hist.at[idxb], add=True)
        return 0

    lax.fori_loop(0, N_FULL, chunk, 0)
    pltpu.sync_copy(dst_hbm.at[pl.ds(base + N_FULL * CHUNK, TAIL)], i16)
    pltpu.sync_copy(ones16, hist.at[i16], add=True)
    plsc.subcore_barrier()
    pltpu.sync_copy(
        hist.at[pl.ds(s * STRIPE, STRIPE)], out_hbm.at[c, pl.ds(s * STRIPE, STRIPE)]
    )


_sc_hist = pl.kernel(
    _hist_body,
    out_type=jax.ShapeDtypeStruct((NC, PAD_N), jnp.float32),
    mesh=_MESH,
    scratch_types=[
        pltpu.VMEM((CHUNK,), jnp.int32),
        pltpu.VMEM((CHUNK,), jnp.float32),
        pltpu.VMEM((TAIL,), jnp.int32),
        pltpu.VMEM((TAIL,), jnp.float32),
        pltpu.VMEM((STRIPE,), jnp.float32),
        pltpu.VMEM_SHARED((PAD_N,), jnp.float32),
    ],
)


def _edge_body(g_hbm, src_hbm, dst_hbm, out_hbm, sidx, didx, rows, s16, d16,
               rows16, zb, acc, sem):
    c = lax.axis_index("c")
    s = lax.axis_index("s")
    wid = s * NC + c
    _fill2d(zb, 64, 0.0)

    def zinit(k, _):
        pltpu.sync_copy(zb, acc.at[pl.ds(s * STRIPE + k * 64, 64)])
        return 0

    lax.fori_loop(0, STRIPE // 64, zinit, 0)
    plsc.subcore_barrier()
    base = wid * E_PER_W

    def chunk(ci, _):
        off = base + ci * CHUNK
        pltpu.sync_copy(src_hbm.at[pl.ds(off, CHUNK)], sidx)
        pltpu.sync_copy(dst_hbm.at[pl.ds(off, CHUNK)], didx)
        pltpu.async_copy(g_hbm.at[sidx], rows, sem).wait()
        pltpu.sync_copy(rows, acc.at[didx], add=True)
        return 0

    lax.fori_loop(0, N_FULL, chunk, 0)
    off = base + N_FULL * CHUNK
    pltpu.sync_copy(src_hbm.at[pl.ds(off, TAIL)], s16)
    pltpu.sync_copy(dst_hbm.at[pl.ds(off, TAIL)], d16)
    pltpu.async_copy(g_hbm.at[s16], rows16, sem).wait()
    pltpu.sync_copy(rows16, acc.at[d16], add=True)
    plsc.subcore_barrier()
    pltpu.sync_copy(
        acc.at[pl.ds(s * STRIPE, STRIPE)], out_hbm.at[c, pl.ds(s * STRIPE, STRIPE)]
    )


_sc_edges = pl.kernel(
    _edge_body,
    out_type=jax.ShapeDtypeStruct((NC, PAD_N, F), jnp.float32),
    mesh=_MESH,
    scratch_types=[
        pltpu.VMEM((CHUNK,), jnp.int32),
        pltpu.VMEM((CHUNK,), jnp.int32),
        pltpu.VMEM((CHUNK, F), jnp.float32),
        pltpu.VMEM((TAIL,), jnp.int32),
        pltpu.VMEM((TAIL,), jnp.int32),
        pltpu.VMEM((TAIL, F), jnp.float32),
        pltpu.VMEM((64, F), jnp.float32),
        pltpu.VMEM_SHARED((PAD_N, F), jnp.float32),
        pltpu.SemaphoreType.DMA,
    ],
)


_TC_BLK = 1280
_TC_GRID = PAD_N // _TC_BLK


def _prep_body(x_ref, w_ref, d0_ref, d1_ref, g_ref, dis_ref):
    deg = d0_ref[...] + d1_ref[...] + 1.0
    dis = lax.rsqrt(deg)
    h = jnp.dot(x_ref[...], w_ref[...], preferred_element_type=jnp.float32)
    g_ref[...] = h * dis
    dis_ref[...] = dis


def _tc_prep(xp, W1, d0, d1):
    return pl.pallas_call(
        _prep_body,
        grid=(_TC_GRID,),
        in_specs=[
            pl.BlockSpec((_TC_BLK, F), lambda i: (i, 0)),
            pl.BlockSpec((F, F), lambda i: (0, 0)),
            pl.BlockSpec((_TC_BLK, 1), lambda i: (i, 0)),
            pl.BlockSpec((_TC_BLK, 1), lambda i: (i, 0)),
        ],
        out_specs=[
            pl.BlockSpec((_TC_BLK, F), lambda i: (i, 0)),
            pl.BlockSpec((_TC_BLK, 1), lambda i: (i, 0)),
        ],
        out_shape=[
            jax.ShapeDtypeStruct((PAD_N, F), jnp.float32),
            jax.ShapeDtypeStruct((PAD_N, 1), jnp.float32),
        ],
    )(xp, W1, d0, d1)


def _final_body(acc_ref, g_ref, dis_ref, b_ref, o_ref):
    t = acc_ref[0] + acc_ref[1] + g_ref[...]
    o_ref[...] = jnp.maximum(t * dis_ref[...] + b_ref[...], 0.0)


def _tc_final(acc, g, dis, b2):
    return pl.pallas_call(
        _final_body,
        grid=(_TC_GRID,),
        in_specs=[
            pl.BlockSpec((NC, _TC_BLK, F), lambda i: (0, i, 0)),
            pl.BlockSpec((_TC_BLK, F), lambda i: (i, 0)),
            pl.BlockSpec((_TC_BLK, 1), lambda i: (i, 0)),
            pl.BlockSpec((1, F), lambda i: (0, 0)),
        ],
        out_specs=pl.BlockSpec((_TC_BLK, F), lambda i: (i, 0)),
        out_shape=jax.ShapeDtypeStruct((PAD_N, F), jnp.float32),
    )(acc, g, dis, b2)


def kernel(x, edge_index, W1, b1):
    src = edge_index[0].astype(jnp.int32)
    dst = edge_index[1].astype(jnp.int32)
    xp = jnp.pad(x, ((0, PAD_N - N), (0, 0)))
    degp = _sc_hist(dst)
    d0 = degp[0].reshape(PAD_N, 1)
    d1 = degp[1].reshape(PAD_N, 1)
    g, dis = _tc_prep(xp, W1, d0, d1)
    acc = _sc_edges(g, src, dst)
    out = _tc_final(acc, g, dis, b1.reshape(1, F))
    return out[:N]
